# baseline (speedup 1.0000x reference)
import functools

import jax
import jax.numpy as jnp
from jax.experimental import pallas as pl
from jax.experimental.pallas import tpu as pltpu

BN_EPS = 1e-5


def _proj_stats_body(x_ref, wq_ref, wk_ref, wv_ref,
                     hq_ref, hk_ref, hv_ref, sq_ref, sk_ref, sv_ref):
    nb = pl.program_id(1)
    x = x_ref[0]
    for w_ref, h_ref, s_ref in ((wq_ref, hq_ref, sq_ref),
                                (wk_ref, hk_ref, sk_ref),
                                (wv_ref, hv_ref, sv_ref)):
        h = jnp.dot(x, w_ref[0], preferred_element_type=jnp.float32)
        h_ref[0] = h
        st = jnp.concatenate(
            [jnp.sum(h, axis=0, keepdims=True),
             jnp.sum(h * h, axis=0, keepdims=True)], axis=0)

        @pl.when(nb == 0)
        def _(s_ref=s_ref, st=st):
            s_ref[0] = st

        @pl.when(nb != 0)
        def _(s_ref=s_ref, st=st):
            s_ref[0] += st


def _attn_body(nbc, hq_ref, hk_ref, hv_ref,
               aq_ref, cq_ref, ak_ref, ck_ref, av_ref, cv_ref,
               o_ref, acc_ref, rs_ref):
    j = pl.program_id(2)
    q = (hq_ref[0] * aq_ref[0] + cq_ref[0]).astype(jnp.bfloat16)
    k = (hk_ref[0] * ak_ref[0] + ck_ref[0]).astype(jnp.bfloat16)
    s = jax.nn.sigmoid(jax.lax.dot_general(
        q, k, (((1,), (1,)), ((), ())), preferred_element_type=jnp.float32))
    w = (hv_ref[0] * av_ref[0] + cv_ref[0]).astype(jnp.bfloat16)

    @pl.when(j == 0)
    def _():
        acc_ref[...] = jnp.zeros_like(acc_ref)
        rs_ref[...] = jnp.zeros_like(rs_ref)

    acc_ref[...] += jnp.dot(s.astype(jnp.bfloat16), w,
                            preferred_element_type=jnp.float32)
    rs_ref[...] += jnp.sum(s, axis=1, keepdims=True)

    @pl.when(j == nbc - 1)
    def _():
        o_ref[0] = acc_ref[...] / (rs_ref[...] + 1e-8)


def kernel(latent_feature, Wq, bq, gq, betaq, Wk, bk, gk, betak, Wv, bv, gv, betav):
    del bq, bk, bv
    V, N, DIN = latent_feature.shape
    DOUT = Wq.shape[-1]

    bma = min(2000, N)
    nba = N // bma
    w_spec = pl.BlockSpec((1, DIN, DOUT), lambda v, nb: (v, 0, 0))
    h_spec = pl.BlockSpec((1, bma, DOUT), lambda v, nb: (v, nb, 0))
    s_spec = pl.BlockSpec((1, 2, DOUT), lambda v, nb: (v, 0, 0))
    h_shape = jax.ShapeDtypeStruct((V, N, DOUT), jnp.float32)
    s_shape = jax.ShapeDtypeStruct((V, 2, DOUT), jnp.float32)
    hq, hk, hv, sq, sk, sv = pl.pallas_call(
        _proj_stats_body,
        grid=(V, nba),
        in_specs=[pl.BlockSpec((1, bma, DIN), lambda v, nb: (v, nb, 0)),
                  w_spec, w_spec, w_spec],
        out_specs=[h_spec, h_spec, h_spec, s_spec, s_spec, s_spec],
        out_shape=[h_shape, h_shape, h_shape, s_shape, s_shape, s_shape],
        compiler_params=pltpu.CompilerParams(
            dimension_semantics=("parallel", "arbitrary")),
        name="proj_stats",
    )(latent_feature, Wq, Wk, Wv)

    def _affine(s, g, beta):
        mean = s[:, 0] / N
        var = s[:, 1] / N - mean * mean
        a = g * jax.lax.rsqrt(var + BN_EPS)
        c = beta - mean * a
        return a[:, None, :], c[:, None, :]

    aq, cq = _affine(sq, gq, betaq)
    ak, ck = _affine(sk, gk, betak)
    av, cv = _affine(sv, gv, betav)

    bm = min(1000, N)
    bk_blk = min(1000, N)
    nbr, nbc = N // bm, N // bk_blk
    p_spec = pl.BlockSpec((1, 1, DOUT), lambda v, i, j: (v, 0, 0))
    out = pl.pallas_call(
        functools.partial(_attn_body, nbc),
        grid=(V, nbr, nbc),
        in_specs=[pl.BlockSpec((1, bm, DOUT), lambda v, i, j: (v, i, 0)),
                  pl.BlockSpec((1, bk_blk, DOUT), lambda v, i, j: (v, j, 0)),
                  pl.BlockSpec((1, bk_blk, DOUT), lambda v, i, j: (v, j, 0)),
                  p_spec, p_spec, p_spec, p_spec, p_spec, p_spec],
        out_specs=pl.BlockSpec((1, bm, DOUT), lambda v, i, j: (v, i, 0)),
        out_shape=jax.ShapeDtypeStruct((V, N, DOUT), jnp.float32),
        scratch_shapes=[pltpu.VMEM((bm, DOUT), jnp.float32),
                        pltpu.VMEM((bm, 1), jnp.float32)],
        compiler_params=pltpu.CompilerParams(
            dimension_semantics=("parallel", "parallel", "arbitrary")),
        name="sigmoid_attn",
    )(hq, hk, hv, aq, cq, ak, ck, av, cv)
    return out


# speedup vs baseline: 1.2277x; 1.2277x over previous
import functools

import jax
import jax.numpy as jnp
from jax.experimental import pallas as pl
from jax.experimental.pallas import tpu as pltpu

BN_EPS = 1e-5


def _proj_stats_body(x_ref, wq_ref, wk_ref, wv_ref,
                     hq_ref, hk_ref, hv_ref, sq_ref, sk_ref, sv_ref):
    nb = pl.program_id(1)
    x = x_ref[0]
    for w_ref, h_ref, s_ref in ((wq_ref, hq_ref, sq_ref),
                                (wk_ref, hk_ref, sk_ref),
                                (wv_ref, hv_ref, sv_ref)):
        h = jnp.dot(x, w_ref[0], preferred_element_type=jnp.float32)
        h_ref[0] = h
        st = jnp.concatenate(
            [jnp.sum(h, axis=0, keepdims=True),
             jnp.sum(h * h, axis=0, keepdims=True)], axis=0)

        @pl.when(nb == 0)
        def _(s_ref=s_ref, st=st):
            s_ref[0] = st

        @pl.when(nb != 0)
        def _(s_ref=s_ref, st=st):
            s_ref[0] += st


def _attn_body(nbc, hq_ref, hk_ref, hv_ref,
               aq_ref, cq_ref, ak_ref, ck_ref, av_ref, cv_ref,
               o_ref, acc_ref):
    j = pl.program_id(2)
    q = (hq_ref[0] * aq_ref[0] + cq_ref[0]).astype(jnp.bfloat16)
    k = (hk_ref[0] * ak_ref[0] + ck_ref[0]).astype(jnp.bfloat16)
    logits = jax.lax.dot_general(
        q, k, (((1,), (1,)), ((), ())), preferred_element_type=jnp.float32)
    s = (jnp.tanh(logits * 0.5) * 0.5 + 0.5).astype(jnp.bfloat16)
    w = (hv_ref[0] * av_ref[0] + cv_ref[0]).astype(jnp.bfloat16)
    w_aug = jnp.concatenate(
        [w, jnp.ones((w.shape[0], 1), jnp.bfloat16)], axis=1)

    @pl.when(j == 0)
    def _():
        acc_ref[...] = jnp.zeros_like(acc_ref)

    acc_ref[...] += jnp.dot(s, w_aug, preferred_element_type=jnp.float32)

    @pl.when(j == nbc - 1)
    def _():
        d = acc_ref[:, -1:]
        o_ref[0] = acc_ref[:, :-1] / (d + 1e-8)


def kernel(latent_feature, Wq, bq, gq, betaq, Wk, bk, gk, betak, Wv, bv, gv, betav):
    del bq, bk, bv
    V, N, DIN = latent_feature.shape
    DOUT = Wq.shape[-1]

    bma = min(2000, N)
    nba = N // bma
    w_spec = pl.BlockSpec((1, DIN, DOUT), lambda v, nb: (v, 0, 0))
    h_spec = pl.BlockSpec((1, bma, DOUT), lambda v, nb: (v, nb, 0))
    s_spec = pl.BlockSpec((1, 2, DOUT), lambda v, nb: (v, 0, 0))
    h_shape = jax.ShapeDtypeStruct((V, N, DOUT), jnp.float32)
    s_shape = jax.ShapeDtypeStruct((V, 2, DOUT), jnp.float32)
    hq, hk, hv, sq, sk, sv = pl.pallas_call(
        _proj_stats_body,
        grid=(V, nba),
        in_specs=[pl.BlockSpec((1, bma, DIN), lambda v, nb: (v, nb, 0)),
                  w_spec, w_spec, w_spec],
        out_specs=[h_spec, h_spec, h_spec, s_spec, s_spec, s_spec],
        out_shape=[h_shape, h_shape, h_shape, s_shape, s_shape, s_shape],
        compiler_params=pltpu.CompilerParams(
            dimension_semantics=("parallel", "arbitrary")),
        name="proj_stats",
    )(latent_feature, Wq, Wk, Wv)

    def _affine(s, g, beta):
        mean = s[:, 0] / N
        var = s[:, 1] / N - mean * mean
        a = g * jax.lax.rsqrt(var + BN_EPS)
        c = beta - mean * a
        return a[:, None, :], c[:, None, :]

    aq, cq = _affine(sq, gq, betaq)
    ak, ck = _affine(sk, gk, betak)
    av, cv = _affine(sv, gv, betav)

    bm = min(1000, N)
    bk_blk = min(1000, N)
    nbr, nbc = N // bm, N // bk_blk
    p_spec = pl.BlockSpec((1, 1, DOUT), lambda v, i, j: (v, 0, 0))
    out = pl.pallas_call(
        functools.partial(_attn_body, nbc),
        grid=(V, nbr, nbc),
        in_specs=[pl.BlockSpec((1, bm, DOUT), lambda v, i, j: (v, i, 0)),
                  pl.BlockSpec((1, bk_blk, DOUT), lambda v, i, j: (v, j, 0)),
                  pl.BlockSpec((1, bk_blk, DOUT), lambda v, i, j: (v, j, 0)),
                  p_spec, p_spec, p_spec, p_spec, p_spec, p_spec],
        out_specs=pl.BlockSpec((1, bm, DOUT), lambda v, i, j: (v, i, 0)),
        out_shape=jax.ShapeDtypeStruct((V, N, DOUT), jnp.float32),
        scratch_shapes=[pltpu.VMEM((bm, DOUT + 1), jnp.float32)],
        compiler_params=pltpu.CompilerParams(
            dimension_semantics=("parallel", "parallel", "arbitrary")),
        name="sigmoid_attn",
    )(hq, hk, hv, aq, cq, ak, ck, av, cv)
    return out


# speedup vs baseline: 1.3082x; 1.0655x over previous
import functools

import jax
import jax.numpy as jnp
from jax.experimental import pallas as pl
from jax.experimental.pallas import tpu as pltpu

BN_EPS = 1e-5


def _proj_stats_body(x_ref, wq_ref, wk_ref, wv_ref,
                     hq_ref, hk_ref, hv_ref, sq_ref, sk_ref, sv_ref):
    nb = pl.program_id(1)
    x = x_ref[0]
    for w_ref, h_ref, s_ref in ((wq_ref, hq_ref, sq_ref),
                                (wk_ref, hk_ref, sk_ref),
                                (wv_ref, hv_ref, sv_ref)):
        h = jnp.dot(x, w_ref[0], preferred_element_type=jnp.float32)
        h_ref[0] = h
        st = jnp.concatenate(
            [jnp.sum(h, axis=0, keepdims=True),
             jnp.sum(h * h, axis=0, keepdims=True)], axis=0)

        @pl.when(nb == 0)
        def _(s_ref=s_ref, st=st):
            s_ref[0] = st

        @pl.when(nb != 0)
        def _(s_ref=s_ref, st=st):
            s_ref[0] += st


def _normalize_body(hq_ref, hk_ref, hv_ref,
                    aq_ref, cq_ref, ak_ref, ck_ref, av_ref, cv_ref,
                    qb_ref, kb_ref, wb_ref):
    qb_ref[0] = (hq_ref[0] * aq_ref[0] + cq_ref[0]).astype(jnp.bfloat16)
    kb_ref[0] = (hk_ref[0] * ak_ref[0] + ck_ref[0]).astype(jnp.bfloat16)
    w = (hv_ref[0] * av_ref[0] + cv_ref[0]).astype(jnp.bfloat16)
    wb_ref[0] = jnp.concatenate(
        [w, jnp.ones((w.shape[0], 1), jnp.bfloat16)], axis=1)


def _attn_body(nbc, qb_ref, kb_ref, wb_ref, csum_ref, o_ref, acc_ref):
    j = pl.program_id(2)
    logits = jax.lax.dot_general(
        qb_ref[0], kb_ref[0], (((1,), (1,)), ((), ())),
        preferred_element_type=jnp.float32)
    t = jnp.tanh(logits).astype(jnp.bfloat16)

    @pl.when(j == 0)
    def _():
        acc_ref[...] = jnp.zeros_like(acc_ref)

    acc_ref[...] += jnp.dot(t, wb_ref[0], preferred_element_type=jnp.float32)

    @pl.when(j == nbc - 1)
    def _():
        n_tot = kb_ref.shape[1] * nbc
        num = acc_ref[:, :-1] + csum_ref[0]
        den = acc_ref[:, -1:] + (n_tot + 2e-8)
        o_ref[0] = num / den


def kernel(latent_feature, Wq, bq, gq, betaq, Wk, bk, gk, betak, Wv, bv, gv, betav):
    del bq, bk, bv
    V, N, DIN = latent_feature.shape
    DOUT = Wq.shape[-1]

    bma = min(2000, N)
    nba = N // bma
    w_spec = pl.BlockSpec((1, DIN, DOUT), lambda v, nb: (v, 0, 0))
    h_spec = pl.BlockSpec((1, bma, DOUT), lambda v, nb: (v, nb, 0))
    s_spec = pl.BlockSpec((1, 2, DOUT), lambda v, nb: (v, 0, 0))
    h_shape = jax.ShapeDtypeStruct((V, N, DOUT), jnp.float32)
    s_shape = jax.ShapeDtypeStruct((V, 2, DOUT), jnp.float32)
    hq, hk, hv, sq, sk, sv = pl.pallas_call(
        _proj_stats_body,
        grid=(V, nba),
        in_specs=[pl.BlockSpec((1, bma, DIN), lambda v, nb: (v, nb, 0)),
                  w_spec, w_spec, w_spec],
        out_specs=[h_spec, h_spec, h_spec, s_spec, s_spec, s_spec],
        out_shape=[h_shape, h_shape, h_shape, s_shape, s_shape, s_shape],
        compiler_params=pltpu.CompilerParams(
            dimension_semantics=("parallel", "arbitrary")),
        name="proj_stats",
    )(latent_feature, Wq, Wk, Wv)

    def _affine(s, g, beta):
        mean = s[:, 0] / N
        var = s[:, 1] / N - mean * mean
        a = g * jax.lax.rsqrt(var + BN_EPS)
        c = beta - mean * a
        return a[:, None, :], c[:, None, :]

    aq, cq = _affine(sq, gq, betaq)
    ak, ck = _affine(sk, gk, betak)
    av, cv = _affine(sv, gv, betav)
    csum = av * sv[:, 0][:, None, :] + N * cv

    p_spec2 = pl.BlockSpec((1, 1, DOUT), lambda v, nb: (v, 0, 0))
    qb, kb, wb = pl.pallas_call(
        _normalize_body,
        grid=(V, nba),
        in_specs=[h_spec, h_spec, h_spec,
                  p_spec2, p_spec2, p_spec2, p_spec2, p_spec2, p_spec2],
        out_specs=[pl.BlockSpec((1, bma, DOUT), lambda v, nb: (v, nb, 0)),
                   pl.BlockSpec((1, bma, DOUT), lambda v, nb: (v, nb, 0)),
                   pl.BlockSpec((1, bma, DOUT + 1), lambda v, nb: (v, nb, 0))],
        out_shape=[jax.ShapeDtypeStruct((V, N, DOUT), jnp.bfloat16),
                   jax.ShapeDtypeStruct((V, N, DOUT), jnp.bfloat16),
                   jax.ShapeDtypeStruct((V, N, DOUT + 1), jnp.bfloat16)],
        compiler_params=pltpu.CompilerParams(
            dimension_semantics=("parallel", "arbitrary")),
        name="normalize",
    )(hq, hk, hv, 0.5 * aq, 0.5 * cq, ak, ck, av, cv)

    bm = min(1000, N)
    bk_blk = min(1000, N)
    nbr, nbc = N // bm, N // bk_blk
    p_spec = pl.BlockSpec((1, 1, DOUT), lambda v, i, j: (v, 0, 0))
    out = pl.pallas_call(
        functools.partial(_attn_body, nbc),
        grid=(V, nbr, nbc),
        in_specs=[pl.BlockSpec((1, bm, DOUT), lambda v, i, j: (v, i, 0)),
                  pl.BlockSpec((1, bk_blk, DOUT), lambda v, i, j: (v, j, 0)),
                  pl.BlockSpec((1, bk_blk, DOUT + 1), lambda v, i, j: (v, j, 0)),
                  p_spec],
        out_specs=pl.BlockSpec((1, bm, DOUT), lambda v, i, j: (v, i, 0)),
        out_shape=jax.ShapeDtypeStruct((V, N, DOUT), jnp.float32),
        scratch_shapes=[pltpu.VMEM((bm, DOUT + 1), jnp.float32)],
        compiler_params=pltpu.CompilerParams(
            dimension_semantics=("parallel", "parallel", "arbitrary")),
        name="sigmoid_attn",
    )(qb, kb, wb, csum)
    return out


# speedup vs baseline: 1.6270x; 1.2437x over previous
import functools

import jax
import jax.numpy as jnp
from jax.experimental import pallas as pl
from jax.experimental.pallas import tpu as pltpu

BN_EPS = 1e-5


def _proj_stats_body(x_ref, wq_ref, wk_ref, wv_ref,
                     hq_ref, hk_ref, hv_ref, sq_ref, sk_ref, sv_ref):
    nb = pl.program_id(1)
    x = x_ref[0]
    for w_ref, h_ref, s_ref in ((wq_ref, hq_ref, sq_ref),
                                (wk_ref, hk_ref, sk_ref),
                                (wv_ref, hv_ref, sv_ref)):
        h = jnp.dot(x, w_ref[0], preferred_element_type=jnp.float32)
        h_ref[0] = h
        st = jnp.concatenate(
            [jnp.sum(h, axis=0, keepdims=True),
             jnp.sum(h * h, axis=0, keepdims=True)], axis=0)

        @pl.when(nb == 0)
        def _(s_ref=s_ref, st=st):
            s_ref[0] = st

        @pl.when(nb != 0)
        def _(s_ref=s_ref, st=st):
            s_ref[0] += st


def _normalize_body(hq_ref, hk_ref, hv_ref,
                    aq_ref, cq_ref, ak_ref, ck_ref, av_ref, cv_ref,
                    qb_ref, kb_ref, wb_ref):
    qb_ref[0] = (hq_ref[0] * aq_ref[0] + cq_ref[0]).astype(jnp.bfloat16)
    kb_ref[0] = (hk_ref[0] * ak_ref[0] + ck_ref[0]).astype(jnp.bfloat16)
    w = (hv_ref[0] * av_ref[0] + cv_ref[0]).astype(jnp.bfloat16)
    wb_ref[0] = jnp.concatenate(
        [w, jnp.ones((w.shape[0], 1), jnp.bfloat16)], axis=1)


def _attn_body(nbc, qb_ref, kb_ref, wb_ref, csum_ref, o_ref, acc_ref):
    j = pl.program_id(2)
    logits = jax.lax.dot_general(
        qb_ref[0], kb_ref[0], (((1,), (1,)), ((), ())),
        preferred_element_type=jnp.float32)
    t = jnp.tanh(logits.astype(jnp.bfloat16))

    @pl.when(j == 0)
    def _():
        acc_ref[...] = jnp.zeros_like(acc_ref)

    acc_ref[...] += jnp.dot(t, wb_ref[0], preferred_element_type=jnp.float32)

    @pl.when(j == nbc - 1)
    def _():
        n_tot = kb_ref.shape[1] * nbc
        num = acc_ref[:, :-1] + csum_ref[0]
        den = acc_ref[:, -1:] + (n_tot + 2e-8)
        o_ref[0] = num / den


def kernel(latent_feature, Wq, bq, gq, betaq, Wk, bk, gk, betak, Wv, bv, gv, betav):
    del bq, bk, bv
    V, N, DIN = latent_feature.shape
    DOUT = Wq.shape[-1]

    bma = min(2000, N)
    nba = N // bma
    w_spec = pl.BlockSpec((1, DIN, DOUT), lambda v, nb: (v, 0, 0))
    h_spec = pl.BlockSpec((1, bma, DOUT), lambda v, nb: (v, nb, 0))
    s_spec = pl.BlockSpec((1, 2, DOUT), lambda v, nb: (v, 0, 0))
    h_shape = jax.ShapeDtypeStruct((V, N, DOUT), jnp.float32)
    s_shape = jax.ShapeDtypeStruct((V, 2, DOUT), jnp.float32)
    hq, hk, hv, sq, sk, sv = pl.pallas_call(
        _proj_stats_body,
        grid=(V, nba),
        in_specs=[pl.BlockSpec((1, bma, DIN), lambda v, nb: (v, nb, 0)),
                  w_spec, w_spec, w_spec],
        out_specs=[h_spec, h_spec, h_spec, s_spec, s_spec, s_spec],
        out_shape=[h_shape, h_shape, h_shape, s_shape, s_shape, s_shape],
        compiler_params=pltpu.CompilerParams(
            dimension_semantics=("parallel", "arbitrary")),
        name="proj_stats",
    )(latent_feature, Wq, Wk, Wv)

    def _affine(s, g, beta):
        mean = s[:, 0] / N
        var = s[:, 1] / N - mean * mean
        a = g * jax.lax.rsqrt(var + BN_EPS)
        c = beta - mean * a
        return a[:, None, :], c[:, None, :]

    aq, cq = _affine(sq, gq, betaq)
    ak, ck = _affine(sk, gk, betak)
    av, cv = _affine(sv, gv, betav)
    csum = av * sv[:, 0][:, None, :] + N * cv

    p_spec2 = pl.BlockSpec((1, 1, DOUT), lambda v, nb: (v, 0, 0))
    qb, kb, wb = pl.pallas_call(
        _normalize_body,
        grid=(V, nba),
        in_specs=[h_spec, h_spec, h_spec,
                  p_spec2, p_spec2, p_spec2, p_spec2, p_spec2, p_spec2],
        out_specs=[pl.BlockSpec((1, bma, DOUT), lambda v, nb: (v, nb, 0)),
                   pl.BlockSpec((1, bma, DOUT), lambda v, nb: (v, nb, 0)),
                   pl.BlockSpec((1, bma, DOUT + 1), lambda v, nb: (v, nb, 0))],
        out_shape=[jax.ShapeDtypeStruct((V, N, DOUT), jnp.bfloat16),
                   jax.ShapeDtypeStruct((V, N, DOUT), jnp.bfloat16),
                   jax.ShapeDtypeStruct((V, N, DOUT + 1), jnp.bfloat16)],
        compiler_params=pltpu.CompilerParams(
            dimension_semantics=("parallel", "arbitrary")),
        name="normalize",
    )(hq, hk, hv, 0.5 * aq, 0.5 * cq, ak, ck, av, cv)

    bm = min(2000, N)
    bk_blk = min(2000, N)
    nbr, nbc = N // bm, N // bk_blk
    p_spec = pl.BlockSpec((1, 1, DOUT), lambda v, i, j: (v, 0, 0))
    out = pl.pallas_call(
        functools.partial(_attn_body, nbc),
        grid=(V, nbr, nbc),
        in_specs=[pl.BlockSpec((1, bm, DOUT), lambda v, i, j: (v, i, 0)),
                  pl.BlockSpec((1, bk_blk, DOUT), lambda v, i, j: (v, j, 0)),
                  pl.BlockSpec((1, bk_blk, DOUT + 1), lambda v, i, j: (v, j, 0)),
                  p_spec],
        out_specs=pl.BlockSpec((1, bm, DOUT), lambda v, i, j: (v, i, 0)),
        out_shape=jax.ShapeDtypeStruct((V, N, DOUT), jnp.float32),
        scratch_shapes=[pltpu.VMEM((bm, DOUT + 1), jnp.float32)],
        compiler_params=pltpu.CompilerParams(
            dimension_semantics=("parallel", "parallel", "arbitrary")),
        name="sigmoid_attn",
    )(qb, kb, wb, csum)
    return out


# speedup vs baseline: 1.7390x; 1.0689x over previous
import functools

import jax
import jax.numpy as jnp
from jax.experimental import pallas as pl
from jax.experimental.pallas import tpu as pltpu

BN_EPS = 1e-5


def _proj_stats_body(x_ref, w_ref, h_ref, s_ref):
    nb = pl.program_id(1)
    h = jnp.dot(x_ref[0], w_ref[0], preferred_element_type=jnp.float32)
    h_ref[0] = h.astype(jnp.bfloat16)
    st = jnp.concatenate(
        [jnp.sum(h, axis=0, keepdims=True),
         jnp.sum(h * h, axis=0, keepdims=True)], axis=0)

    @pl.when(nb == 0)
    def _():
        s_ref[0] = st

    @pl.when(nb != 0)
    def _():
        s_ref[0] += st


def _normalize_body(dout, h_ref, a_ref, c_ref, qb_ref, kb_ref, wb_ref):
    u = h_ref[0] * a_ref[0] + c_ref[0]
    qb_ref[0] = u[:, :dout].astype(jnp.bfloat16)
    kb_ref[0] = u[:, dout:2 * dout].astype(jnp.bfloat16)
    w = u[:, 2 * dout:].astype(jnp.bfloat16)
    wb_ref[0] = jnp.concatenate(
        [w, jnp.ones((w.shape[0], 1), jnp.bfloat16)], axis=1)


def _attn_body(nbc, qb_ref, kb_ref, wb_ref, csum_ref, o_ref, acc_ref):
    j = pl.program_id(2)
    logits = jax.lax.dot_general(
        qb_ref[0], kb_ref[0], (((1,), (1,)), ((), ())),
        preferred_element_type=jnp.float32)
    t = jnp.tanh(logits.astype(jnp.bfloat16))

    @pl.when(j == 0)
    def _():
        acc_ref[...] = jnp.zeros_like(acc_ref)

    acc_ref[...] += jnp.dot(t, wb_ref[0], preferred_element_type=jnp.float32)

    @pl.when(j == nbc - 1)
    def _():
        n_tot = kb_ref.shape[1] * nbc
        num = acc_ref[:, :-1] + csum_ref[0]
        den = acc_ref[:, -1:] + (n_tot + 2e-8)
        o_ref[0] = num / den


def kernel(latent_feature, Wq, bq, gq, betaq, Wk, bk, gk, betak, Wv, bv, gv, betav):
    del bq, bk, bv
    V, N, DIN = latent_feature.shape
    DOUT = Wq.shape[-1]
    D3 = 3 * DOUT

    w_cat = jnp.concatenate([Wq, Wk, Wv], axis=-1)

    bma = min(2000, N)
    nba = N // bma
    h_cat, s_cat = pl.pallas_call(
        _proj_stats_body,
        grid=(V, nba),
        in_specs=[pl.BlockSpec((1, bma, DIN), lambda v, nb: (v, nb, 0)),
                  pl.BlockSpec((1, DIN, D3), lambda v, nb: (v, 0, 0))],
        out_specs=[pl.BlockSpec((1, bma, D3), lambda v, nb: (v, nb, 0)),
                   pl.BlockSpec((1, 2, D3), lambda v, nb: (v, 0, 0))],
        out_shape=[jax.ShapeDtypeStruct((V, N, D3), jnp.bfloat16),
                   jax.ShapeDtypeStruct((V, 2, D3), jnp.float32)],
        compiler_params=pltpu.CompilerParams(
            dimension_semantics=("parallel", "arbitrary")),
        name="proj_stats",
    )(latent_feature, w_cat)

    g_cat = jnp.concatenate([gq, gk, gv], axis=-1)
    beta_cat = jnp.concatenate([betaq, betak, betav], axis=-1)
    mean = s_cat[:, 0] / N
    var = s_cat[:, 1] / N - mean * mean
    a_cat = g_cat * jax.lax.rsqrt(var + BN_EPS)
    c_cat = beta_cat - mean * a_cat
    scale = jnp.concatenate([jnp.full((DOUT,), 0.5, jnp.float32),
                             jnp.ones((2 * DOUT,), jnp.float32)])
    a_cat = (a_cat * scale)[:, None, :]
    c_cat = (c_cat * scale)[:, None, :]
    av = a_cat[:, :, 2 * DOUT:]
    cv = c_cat[:, :, 2 * DOUT:]
    csum = av * s_cat[:, 0][:, None, 2 * DOUT:] + N * cv

    p_cat = pl.BlockSpec((1, 1, D3), lambda v, nb: (v, 0, 0))
    qb, kb, wb = pl.pallas_call(
        functools.partial(_normalize_body, DOUT),
        grid=(V, nba),
        in_specs=[pl.BlockSpec((1, bma, D3), lambda v, nb: (v, nb, 0)),
                  p_cat, p_cat],
        out_specs=[pl.BlockSpec((1, bma, DOUT), lambda v, nb: (v, nb, 0)),
                   pl.BlockSpec((1, bma, DOUT), lambda v, nb: (v, nb, 0)),
                   pl.BlockSpec((1, bma, DOUT + 1), lambda v, nb: (v, nb, 0))],
        out_shape=[jax.ShapeDtypeStruct((V, N, DOUT), jnp.bfloat16),
                   jax.ShapeDtypeStruct((V, N, DOUT), jnp.bfloat16),
                   jax.ShapeDtypeStruct((V, N, DOUT + 1), jnp.bfloat16)],
        compiler_params=pltpu.CompilerParams(
            dimension_semantics=("parallel", "arbitrary")),
        name="normalize",
    )(h_cat, a_cat, c_cat)

    bm = min(2000, N)
    bk_blk = min(2000, N)
    nbr, nbc = N // bm, N // bk_blk
    p_spec = pl.BlockSpec((1, 1, DOUT), lambda v, i, j: (v, 0, 0))
    out = pl.pallas_call(
        functools.partial(_attn_body, nbc),
        grid=(V, nbr, nbc),
        in_specs=[pl.BlockSpec((1, bm, DOUT), lambda v, i, j: (v, i, 0)),
                  pl.BlockSpec((1, bk_blk, DOUT), lambda v, i, j: (v, j, 0)),
                  pl.BlockSpec((1, bk_blk, DOUT + 1), lambda v, i, j: (v, j, 0)),
                  p_spec],
        out_specs=pl.BlockSpec((1, bm, DOUT), lambda v, i, j: (v, i, 0)),
        out_shape=jax.ShapeDtypeStruct((V, N, DOUT), jnp.float32),
        scratch_shapes=[pltpu.VMEM((bm, DOUT + 1), jnp.float32)],
        compiler_params=pltpu.CompilerParams(
            dimension_semantics=("parallel", "parallel", "arbitrary")),
        name="sigmoid_attn",
    )(qb, kb, wb, csum)
    return out
